# baseline (speedup 1.0000x reference)
import functools

import jax
import jax.numpy as jnp
from jax import lax
from jax.experimental import pallas as pl
from jax.experimental.pallas import tpu as pltpu
from jax.experimental.pallas import tpu_sc as plsc

_B, _D, _N = 8, 3, 4096
_K, _KP1 = 16, 17
_Q = 512
_NT = 2048
_NC = 2
_NW = 32
_WPB = _NW // _B
_JW = _N // _WPB


def _knn_body(pct_ref, pcq_ref, idx_ref):
    pcT = pct_ref[0]
    qc = pcq_ref[0]
    r2 = jnp.sum(pcT * pcT, axis=1, keepdims=True)
    q2 = jnp.sum(qc * qc, axis=0, keepdims=True)
    inner = (pcT[:, 0:1] * qc[0:1, :]
             + pcT[:, 1:2] * qc[1:2, :]
             + pcT[:, 2:3] * qc[2:3, :])
    run = (r2 + q2) - 2.0 * inner
    iota = lax.broadcasted_iota(jnp.int32, (_N, _Q), 0)
    inf = jnp.float32(jnp.inf)
    m = jnp.min(run, axis=0, keepdims=True)
    for t in range(_KP1):
        cand = jnp.where(run == m, iota, _N)
        amin = jnp.min(cand, axis=0, keepdims=True)
        idx_ref[0, t, :] = amin[0]
        if t + 1 < _KP1:
            run = jnp.where(iota == amin, inf, run)
            m = jnp.min(run, axis=0, keepdims=True)


def _knn_topk(pc_t, pc):
    return pl.pallas_call(
        _knn_body,
        grid=(_B, _N // _Q),
        in_specs=[
            pl.BlockSpec((1, _N, _D), lambda b, q: (b, 0, 0)),
            pl.BlockSpec((1, _D, _Q), lambda b, q: (b, 0, q)),
        ],
        out_specs=pl.BlockSpec((1, _KP1, _Q), lambda b, q: (b, 0, q)),
        out_shape=jax.ShapeDtypeStruct((_B, _KP1, _N), jnp.int32),
    )(pc_t, pc)


def _gather_body(pc_flat_hbm, idx_hbm, nb_hbm, table_v, idx_v, nb_v):
    wid = lax.axis_index("s") * _NC + lax.axis_index("c")
    b = wid // _WPB
    j0 = (wid % _WPB) * _JW
    pltpu.sync_copy(pc_flat_hbm.at[b], table_v)
    pltpu.sync_copy(idx_hbm.at[b, :, pl.ds(j0, _JW)], idx_v)

    def body(k, carry):
        i = k // (_JW // 16)
        g = k % (_JW // 16)
        vidx = idx_v[i, pl.ds(g * 16, 16)]
        for c in range(_D):
            vals = plsc.load_gather(table_v, [vidx + c * _N])
            nb_v[c, i, pl.ds(g * 16, 16)] = vals
        return carry

    lax.fori_loop(0, _K * (_JW // 16), body, 0)
    pltpu.sync_copy(nb_v, nb_hbm.at[b, :, :, pl.ds(j0, _JW)])


def _gather(pc_flat, idx):
    mesh = plsc.VectorSubcoreMesh(core_axis_name="c", subcore_axis_name="s")
    k = functools.partial(
        pl.kernel,
        mesh=mesh,
        out_type=jax.ShapeDtypeStruct((_B, _D, _K, _N), jnp.float32),
        scratch_types=[
            pltpu.VMEM((_D * _N,), jnp.float32),
            pltpu.VMEM((_K, _JW), jnp.int32),
            pltpu.VMEM((_D, _K, _JW), jnp.float32),
        ],
        compiler_params=pltpu.CompilerParams(needs_layout_passes=False),
    )(_gather_body)
    return k(pc_flat, idx)


def _edge_body(inp_ref, nb_ref, out_ref):
    c2 = pl.program_id(1)
    inp = inp_ref[0, 0]
    nb = nb_ref[0, 0]

    @pl.when(c2 < _D)
    def _():
        out_ref[0, 0] = jnp.broadcast_to(inp[None, :, :], (_K, _D, _NT))

    @pl.when(c2 >= _D)
    def _():
        out_ref[0, 0] = nb[:, None, :] - inp[None, :, :]


def _edge(inp, neighbors):
    return pl.pallas_call(
        _edge_body,
        grid=(_B, 2 * _D, _N // _NT),
        in_specs=[
            pl.BlockSpec((1, 1, _D, _NT),
                         lambda b, c2, n: (b, c2 % _D, 0, n)),
            pl.BlockSpec((1, 1, _K, _NT),
                         lambda b, c2, n: (b, c2 % _D, 0, n)),
        ],
        out_specs=pl.BlockSpec((1, 1, _K, _D, _NT),
                               lambda b, c2, n: (b, c2, 0, 0, n)),
        out_shape=jax.ShapeDtypeStruct((_B, 2 * _D, _K, _D, _N), jnp.float32),
    )(inp, neighbors)


def kernel(point_cloud, input):
    pc = point_cloud
    pc_t = jnp.transpose(pc, (0, 2, 1))
    idx17 = _knn_topk(pc_t, pc)
    idx = lax.slice(idx17, (0, 1, 0), (_B, _KP1, _N))
    pc_flat = pc.reshape(_B, _D * _N)
    neighbors = _gather(pc_flat, idx)
    edge = _edge(input, neighbors)
    return edge, idx


# speedup vs baseline: 1.2230x; 1.2230x over previous
import functools

import jax
import jax.numpy as jnp
from jax import lax
from jax.experimental import pallas as pl
from jax.experimental.pallas import tpu as pltpu
from jax.experimental.pallas import tpu_sc as plsc

_B, _D, _N = 8, 3, 4096
_K, _KP1 = 16, 17
_Q = 1024
_NT = 2048
_NC = 2
_NW = 32
_WPB = _NW // _B
_JW = _N // _WPB


def _knn_body(pct_ref, pcq_ref, idx_ref):
    pcT = pct_ref[0]
    qc = pcq_ref[0]
    r2 = jnp.sum(pcT * pcT, axis=1, keepdims=True)
    q2 = jnp.sum(qc * qc, axis=0, keepdims=True)
    inner = lax.dot_general(pcT, qc, (((1,), (0,)), ((), ())),
                            preferred_element_type=jnp.float32)
    run = (r2 + q2) - 2.0 * inner
    iota = lax.broadcasted_iota(jnp.int32, (_N, _Q), 0)
    inf = jnp.float32(jnp.inf)
    m = jnp.min(run, axis=0, keepdims=True)
    for t in range(_KP1):
        cand = jnp.where(run == m, iota, _N)
        amin = jnp.min(cand, axis=0, keepdims=True)
        idx_ref[0, t, :] = amin[0]
        if t + 1 < _KP1:
            run = jnp.where(iota == amin, inf, run)
            m = jnp.min(run, axis=0, keepdims=True)


def _knn_topk(pc_t, pc):
    return pl.pallas_call(
        _knn_body,
        grid=(_B, _N // _Q),
        in_specs=[
            pl.BlockSpec((1, _N, _D), lambda b, q: (b, 0, 0)),
            pl.BlockSpec((1, _D, _Q), lambda b, q: (b, 0, q)),
        ],
        out_specs=pl.BlockSpec((1, _KP1, _Q), lambda b, q: (b, 0, q)),
        out_shape=jax.ShapeDtypeStruct((_B, _KP1, _N), jnp.int32),
    )(pc_t, pc)


def _gather_body(pc_flat_hbm, idx_hbm, nb_hbm, table_v, idx_v, nb_v):
    wid = lax.axis_index("s") * _NC + lax.axis_index("c")
    b = wid // _WPB
    j0 = (wid % _WPB) * _JW
    pltpu.sync_copy(pc_flat_hbm.at[b], table_v)
    pltpu.sync_copy(idx_hbm.at[b, :, pl.ds(j0, _JW)], idx_v)

    def body(k, carry):
        i = k // (_JW // 16)
        g = k % (_JW // 16)
        vidx = idx_v[i, pl.ds(g * 16, 16)]
        for c in range(_D):
            vals = plsc.load_gather(table_v, [vidx + c * _N])
            nb_v[c, i, pl.ds(g * 16, 16)] = vals
        return carry

    lax.fori_loop(0, _K * (_JW // 16), body, 0)
    pltpu.sync_copy(nb_v, nb_hbm.at[b, :, :, pl.ds(j0, _JW)])


def _gather(pc_flat, idx):
    mesh = plsc.VectorSubcoreMesh(core_axis_name="c", subcore_axis_name="s")
    k = functools.partial(
        pl.kernel,
        mesh=mesh,
        out_type=jax.ShapeDtypeStruct((_B, _D, _K, _N), jnp.float32),
        scratch_types=[
            pltpu.VMEM((_D * _N,), jnp.float32),
            pltpu.VMEM((_K, _JW), jnp.int32),
            pltpu.VMEM((_D, _K, _JW), jnp.float32),
        ],
        compiler_params=pltpu.CompilerParams(needs_layout_passes=False),
    )(_gather_body)
    return k(pc_flat, idx)


def _edge_body(inp_ref, nb_ref, out_ref):
    c2 = pl.program_id(1)
    inp = inp_ref[0, 0]
    nb = nb_ref[0, 0]

    @pl.when(c2 < _D)
    def _():
        out_ref[0, 0] = jnp.broadcast_to(inp[None, :, :], (_K, _D, _NT))

    @pl.when(c2 >= _D)
    def _():
        out_ref[0, 0] = nb[:, None, :] - inp[None, :, :]


def _edge(inp, neighbors):
    return pl.pallas_call(
        _edge_body,
        grid=(_B, 2 * _D, _N // _NT),
        in_specs=[
            pl.BlockSpec((1, 1, _D, _NT),
                         lambda b, c2, n: (b, c2 % _D, 0, n)),
            pl.BlockSpec((1, 1, _K, _NT),
                         lambda b, c2, n: (b, c2 % _D, 0, n)),
        ],
        out_specs=pl.BlockSpec((1, 1, _K, _D, _NT),
                               lambda b, c2, n: (b, c2, 0, 0, n)),
        out_shape=jax.ShapeDtypeStruct((_B, 2 * _D, _K, _D, _N), jnp.float32),
    )(inp, neighbors)


def kernel(point_cloud, input):
    pc = point_cloud
    pc_t = jnp.transpose(pc, (0, 2, 1))
    idx17 = _knn_topk(pc_t, pc)
    idx = lax.slice(idx17, (0, 1, 0), (_B, _KP1, _N))
    pc_flat = pc.reshape(_B, _D * _N)
    neighbors = _gather(pc_flat, idx)
    edge = _edge(input, neighbors)
    return edge, idx


# speedup vs baseline: 1.3050x; 1.0671x over previous
import functools

import jax
import jax.numpy as jnp
from jax import lax
from jax.experimental import pallas as pl
from jax.experimental.pallas import tpu as pltpu
from jax.experimental.pallas import tpu_sc as plsc

_B, _D, _N = 8, 3, 4096
_K, _KP1 = 16, 17
_Q = 2048
_NT = 2048
_NC = 2
_NW = 32
_WPB = _NW // _B
_JW = _N // _WPB


def _knn_body(pct_ref, pcq_ref, idx_ref):
    pcT = pct_ref[0]
    qc = pcq_ref[0]
    r2 = jnp.sum(pcT * pcT, axis=1, keepdims=True)
    q2 = jnp.sum(qc * qc, axis=0, keepdims=True)
    inner = lax.dot_general(pcT, qc, (((1,), (0,)), ((), ())),
                            preferred_element_type=jnp.float32)
    run = (r2 + q2) - 2.0 * inner
    iota = lax.broadcasted_iota(jnp.int32, (_N, _Q), 0)
    inf = jnp.float32(jnp.inf)
    m = jnp.min(run, axis=0, keepdims=True)
    for t in range(_KP1):
        cand = jnp.where(run == m, iota, _N)
        amin = jnp.min(cand, axis=0, keepdims=True)
        idx_ref[0, t, :] = amin[0]
        if t + 1 < _KP1:
            run = jnp.where(iota == amin, inf, run)
            m = jnp.min(run, axis=0, keepdims=True)


def _knn_topk(pc_t, pc):
    return pl.pallas_call(
        _knn_body,
        grid=(_B, _N // _Q),
        in_specs=[
            pl.BlockSpec((1, _N, _D), lambda b, q: (b, 0, 0)),
            pl.BlockSpec((1, _D, _Q), lambda b, q: (b, 0, q)),
        ],
        out_specs=pl.BlockSpec((1, _KP1, _Q), lambda b, q: (b, 0, q)),
        out_shape=jax.ShapeDtypeStruct((_B, _KP1, _N), jnp.int32),
    )(pc_t, pc)


def _gather_body(pc_flat_hbm, idx_hbm, nb_hbm, table_v, idx_v, nb_v):
    wid = lax.axis_index("s") * _NC + lax.axis_index("c")
    b = wid // _WPB
    j0 = (wid % _WPB) * _JW
    pltpu.sync_copy(pc_flat_hbm.at[b], table_v)
    pltpu.sync_copy(idx_hbm.at[b, :, pl.ds(j0, _JW)], idx_v)

    def body(k, carry):
        i = k // (_JW // 16)
        g = k % (_JW // 16)
        vidx = idx_v[i, pl.ds(g * 16, 16)]
        for c in range(_D):
            vals = plsc.load_gather(table_v, [vidx + c * _N])
            nb_v[c, i, pl.ds(g * 16, 16)] = vals
        return carry

    lax.fori_loop(0, _K * (_JW // 16), body, 0)
    pltpu.sync_copy(nb_v, nb_hbm.at[b, :, :, pl.ds(j0, _JW)])


def _gather(pc_flat, idx):
    mesh = plsc.VectorSubcoreMesh(core_axis_name="c", subcore_axis_name="s")
    k = functools.partial(
        pl.kernel,
        mesh=mesh,
        out_type=jax.ShapeDtypeStruct((_B, _D, _K, _N), jnp.float32),
        scratch_types=[
            pltpu.VMEM((_D * _N,), jnp.float32),
            pltpu.VMEM((_K, _JW), jnp.int32),
            pltpu.VMEM((_D, _K, _JW), jnp.float32),
        ],
        compiler_params=pltpu.CompilerParams(needs_layout_passes=False),
    )(_gather_body)
    return k(pc_flat, idx)


def _edge_body(inp_ref, nb_ref, out_ref):
    c2 = pl.program_id(1)
    inp = inp_ref[0, 0]
    nb = nb_ref[0, 0]

    @pl.when(c2 < _D)
    def _():
        out_ref[0, 0] = jnp.broadcast_to(inp[None, :, :], (_K, _D, _NT))

    @pl.when(c2 >= _D)
    def _():
        out_ref[0, 0] = nb[:, None, :] - inp[None, :, :]


def _edge(inp, neighbors):
    return pl.pallas_call(
        _edge_body,
        grid=(_B, 2 * _D, _N // _NT),
        in_specs=[
            pl.BlockSpec((1, 1, _D, _NT),
                         lambda b, c2, n: (b, c2 % _D, 0, n)),
            pl.BlockSpec((1, 1, _K, _NT),
                         lambda b, c2, n: (b, c2 % _D, 0, n)),
        ],
        out_specs=pl.BlockSpec((1, 1, _K, _D, _NT),
                               lambda b, c2, n: (b, c2, 0, 0, n)),
        out_shape=jax.ShapeDtypeStruct((_B, 2 * _D, _K, _D, _N), jnp.float32),
    )(inp, neighbors)


def kernel(point_cloud, input):
    pc = point_cloud
    pc_t = jnp.transpose(pc, (0, 2, 1))
    idx17 = _knn_topk(pc_t, pc)
    idx = lax.slice(idx17, (0, 1, 0), (_B, _KP1, _N))
    pc_flat = pc.reshape(_B, _D * _N)
    neighbors = _gather(pc_flat, idx)
    edge = _edge(input, neighbors)
    return edge, idx


# speedup vs baseline: 1.3509x; 1.0352x over previous
import functools

import jax
import jax.numpy as jnp
from jax import lax
from jax.experimental import pallas as pl
from jax.experimental.pallas import tpu as pltpu
from jax.experimental.pallas import tpu_sc as plsc

_B, _D, _N = 8, 3, 4096
_K, _KP1 = 16, 17
_Q = 4096
_NT = 2048
_NC = 2
_NW = 32
_WPB = _NW // _B
_JW = _N // _WPB


def _knn_body(pct_ref, pcq_ref, idx_ref):
    pcT = pct_ref[0]
    qc = pcq_ref[0]
    r2 = jnp.sum(pcT * pcT, axis=1, keepdims=True)
    q2 = jnp.sum(qc * qc, axis=0, keepdims=True)
    inner = lax.dot_general(pcT, qc, (((1,), (0,)), ((), ())),
                            preferred_element_type=jnp.float32)
    run = (r2 + q2) - 2.0 * inner
    iota = lax.broadcasted_iota(jnp.int32, (_N, _Q), 0)
    inf = jnp.float32(jnp.inf)
    m = jnp.min(run, axis=0, keepdims=True)
    for t in range(_KP1):
        cand = jnp.where(run == m, iota, _N)
        amin = jnp.min(cand, axis=0, keepdims=True)
        idx_ref[0, t, :] = amin[0]
        if t + 1 < _KP1:
            run = jnp.where(iota == amin, inf, run)
            m = jnp.min(run, axis=0, keepdims=True)


def _knn_topk(pc_t, pc):
    return pl.pallas_call(
        _knn_body,
        grid=(_B, _N // _Q),
        in_specs=[
            pl.BlockSpec((1, _N, _D), lambda b, q: (b, 0, 0)),
            pl.BlockSpec((1, _D, _Q), lambda b, q: (b, 0, q)),
        ],
        out_specs=pl.BlockSpec((1, _KP1, _Q), lambda b, q: (b, 0, q)),
        out_shape=jax.ShapeDtypeStruct((_B, _KP1, _N), jnp.int32),
    )(pc_t, pc)


def _gather_body(pc_flat_hbm, idx_hbm, nb_hbm, table_v, idx_v, nb_v):
    wid = lax.axis_index("s") * _NC + lax.axis_index("c")
    b = wid // _WPB
    j0 = (wid % _WPB) * _JW
    pltpu.sync_copy(pc_flat_hbm.at[b], table_v)
    pltpu.sync_copy(idx_hbm.at[b, :, pl.ds(j0, _JW)], idx_v)

    def body(k, carry):
        i = k // (_JW // 16)
        g = k % (_JW // 16)
        vidx = idx_v[i, pl.ds(g * 16, 16)]
        for c in range(_D):
            vals = plsc.load_gather(table_v, [vidx + c * _N])
            nb_v[c, i, pl.ds(g * 16, 16)] = vals
        return carry

    lax.fori_loop(0, _K * (_JW // 16), body, 0)
    pltpu.sync_copy(nb_v, nb_hbm.at[b, :, :, pl.ds(j0, _JW)])


def _gather(pc_flat, idx):
    mesh = plsc.VectorSubcoreMesh(core_axis_name="c", subcore_axis_name="s")
    k = functools.partial(
        pl.kernel,
        mesh=mesh,
        out_type=jax.ShapeDtypeStruct((_B, _D, _K, _N), jnp.float32),
        scratch_types=[
            pltpu.VMEM((_D * _N,), jnp.float32),
            pltpu.VMEM((_K, _JW), jnp.int32),
            pltpu.VMEM((_D, _K, _JW), jnp.float32),
        ],
        compiler_params=pltpu.CompilerParams(needs_layout_passes=False),
    )(_gather_body)
    return k(pc_flat, idx)


def _edge_body(inp_ref, nb_ref, out_ref):
    c2 = pl.program_id(1)
    inp = inp_ref[0, 0]
    nb = nb_ref[0, 0]

    @pl.when(c2 < _D)
    def _():
        out_ref[0, 0] = jnp.broadcast_to(inp[None, :, :], (_K, _D, _NT))

    @pl.when(c2 >= _D)
    def _():
        out_ref[0, 0] = nb[:, None, :] - inp[None, :, :]


def _edge(inp, neighbors):
    return pl.pallas_call(
        _edge_body,
        grid=(_B, 2 * _D, _N // _NT),
        in_specs=[
            pl.BlockSpec((1, 1, _D, _NT),
                         lambda b, c2, n: (b, c2 % _D, 0, n)),
            pl.BlockSpec((1, 1, _K, _NT),
                         lambda b, c2, n: (b, c2 % _D, 0, n)),
        ],
        out_specs=pl.BlockSpec((1, 1, _K, _D, _NT),
                               lambda b, c2, n: (b, c2, 0, 0, n)),
        out_shape=jax.ShapeDtypeStruct((_B, 2 * _D, _K, _D, _N), jnp.float32),
    )(inp, neighbors)


def kernel(point_cloud, input):
    pc = point_cloud
    pc_t = jnp.transpose(pc, (0, 2, 1))
    idx17 = _knn_topk(pc_t, pc)
    idx = lax.slice(idx17, (0, 1, 0), (_B, _KP1, _N))
    pc_flat = pc.reshape(_B, _D * _N)
    neighbors = _gather(pc_flat, idx)
    edge = _edge(input, neighbors)
    return edge, idx
